# baseline (speedup 1.0000x reference)
import functools

import jax
import jax.numpy as jnp
from jax import lax
from jax.experimental import pallas as pl
from jax.experimental.pallas import tpu as pltpu
from jax.experimental.pallas import tpu_sc as plsc

N = 10000
E = 320000
IN_DIM = 128
NUM_CONFS = 16
CONF_EMB = 4
HIDDEN = 64
OUT = 32

NPAD = 10240
NCORES = 2
NSUB = 16
NW = NCORES * NSUB
CHUNK = 128
EPW = E // NW
FULL = EPW // CHUNK
TAIL = EPW - FULL * CHUNK
SIDX = 10016
ROWS_PER_TILE = NPAD // NSUB

_mesh = lambda: plsc.VectorSubcoreMesh(core_axis_name="c", subcore_axis_name="s")
_SC_PARAMS = pltpu.CompilerParams(use_tc_tiling_on_sc=False)


@functools.partial(
    pl.kernel,
    out_type=jax.ShapeDtypeStruct((NCORES, NPAD, 1), jnp.float32),
    mesh=_mesh(),
    compiler_params=_SC_PARAMS,
    scratch_types=[
        pltpu.VMEM_SHARED((NPAD, 1), jnp.float32),
        pltpu.VMEM((CHUNK,), jnp.int32),
        pltpu.VMEM((CHUNK,), jnp.int32),
        pltpu.VMEM((TAIL,), jnp.int32),
        pltpu.VMEM((CHUNK, 1), jnp.float32),
        pltpu.VMEM((TAIL, 1), jnp.float32),
        pltpu.SemaphoreType.DMA,
        pltpu.SemaphoreType.DMA,
    ],
)
def _sc_deg(ei_hbm, ones_hbm, out_hbm,
            deg_s, dA, dB, dT, ones_v, ones_t, semA, semB):
    c = lax.axis_index("c")
    s = lax.axis_index("s")
    w = c * NSUB + s
    r0 = s * ROWS_PER_TILE
    base = w * EPW

    pltpu.sync_copy(ones_hbm, ones_v)
    pltpu.sync_copy(ones_hbm.at[pl.ds(0, TAIL)], ones_t)
    for i in range(ROWS_PER_TILE // CHUNK):
        pltpu.sync_copy(ones_v, deg_s.at[pl.ds(r0 + i * CHUNK, CHUNK)])

    plsc.subcore_barrier()

    @pl.loop(0, FULL // 2)
    def _(j):
        o = base + j * (2 * CHUNK)
        cA = pltpu.async_copy(ei_hbm.at[1, pl.ds(o, CHUNK)], dA, semA)
        cB = pltpu.async_copy(ei_hbm.at[1, pl.ds(o + CHUNK, CHUNK)], dB, semB)
        cA.wait()
        pltpu.sync_copy(ones_v, deg_s.at[dA], add=True)
        cB.wait()
        pltpu.sync_copy(ones_v, deg_s.at[dB], add=True)

    pltpu.sync_copy(ei_hbm.at[1, pl.ds(base + FULL * CHUNK, TAIL)], dT)
    pltpu.sync_copy(ones_t, deg_s.at[dT], add=True)

    plsc.subcore_barrier()
    pltpu.sync_copy(deg_s.at[pl.ds(r0, ROWS_PER_TILE)],
                    out_hbm.at[c, pl.ds(r0, ROWS_PER_TILE)])


NBUF = 4
NGROUPS = FULL // NBUF
REM = FULL - NGROUPS * NBUF


def _make_sc_agg(D):
    @functools.partial(
        pl.kernel,
        out_type=jax.ShapeDtypeStruct((NCORES, NPAD, D), jnp.float32),
        mesh=_mesh(),
        compiler_params=_SC_PARAMS,
        scratch_types=(
            [pltpu.VMEM_SHARED((NPAD, D), jnp.float32),
             pltpu.VMEM_SHARED((NPAD, D), jnp.float32),
             pltpu.VMEM((SIDX,), jnp.int32),
             pltpu.VMEM((TAIL,), jnp.int32),
             pltpu.VMEM((TAIL, D), jnp.float32)]
            + [pltpu.VMEM((CHUNK,), jnp.int32)] * NBUF
            + [pltpu.VMEM((CHUNK, D), jnp.float32)] * NBUF
            + [pltpu.SemaphoreType.DMA] * (3 * NBUF)
        ),
    )
    def sc_agg(g_hbm, ei_hbm, out_hbm, agg_s, gtab_s, sidx, dtail, rtail,
               *bufs):
        dbufs = bufs[0:NBUF]
        rows = bufs[NBUF:2 * NBUF]
        semd = bufs[2 * NBUF:3 * NBUF]
        semg = bufs[3 * NBUF:4 * NBUF]
        sems = bufs[4 * NBUF:5 * NBUF]
        c = lax.axis_index("c")
        s = lax.axis_index("s")
        w = c * NSUB + s
        r0 = s * ROWS_PER_TILE
        base = w * EPW

        pltpu.sync_copy(ei_hbm.at[0, pl.ds(base, EPW)],
                        sidx.at[pl.ds(0, EPW)])
        pltpu.sync_copy(g_hbm.at[pl.ds(r0, ROWS_PER_TILE)],
                        gtab_s.at[pl.ds(r0, ROWS_PER_TILE)])
        pltpu.sync_copy(g_hbm.at[pl.ds(r0, ROWS_PER_TILE)],
                        agg_s.at[pl.ds(r0, ROWS_PER_TILE)])

        plsc.subcore_barrier()

        def do_group(j, nbuf):
            gds = []
            dds = []
            for k in range(nbuf):
                o = base + (j * NBUF + k) * CHUNK
                dds.append(pltpu.async_copy(ei_hbm.at[1, pl.ds(o, CHUNK)],
                                            dbufs[k], semd[k]))
                gds.append(pltpu.async_copy(
                    gtab_s.at[sidx.at[pl.ds((j * NBUF + k) * CHUNK, CHUNK)]],
                    rows[k], semg[k]))
            sds = []
            for k in range(nbuf):
                dds[k].wait()
                gds[k].wait()
                sds.append(pltpu.async_copy(rows[k], agg_s.at[dbufs[k]],
                                            sems[k], add=True))
            for d in sds:
                d.wait()

        @pl.loop(0, NGROUPS)
        def _(j):
            do_group(j, NBUF)

        do_group(NGROUPS, REM)

        pltpu.sync_copy(ei_hbm.at[1, pl.ds(base + FULL * CHUNK, TAIL)], dtail)
        pltpu.sync_copy(gtab_s.at[sidx.at[pl.ds(FULL * CHUNK, TAIL)]], rtail)
        pltpu.sync_copy(rtail, agg_s.at[dtail], add=True)

        plsc.subcore_barrier()
        pltpu.sync_copy(agg_s.at[pl.ds(r0, ROWS_PER_TILE)],
                        out_hbm.at[c, pl.ds(r0, ROWS_PER_TILE)])

    return sc_agg


_sc_agg64 = _make_sc_agg(HIDDEN)
_sc_agg32 = _make_sc_agg(OUT)


RBP = NPAD
GRIDP = 1


def _tc1_body(x_ref, cid_ref, ct_ref, w1a_ref, w1b_ref, degp_ref,
              g1_ref, dinv_ref):
    xb = x_ref[...]
    ids = cid_ref[...]
    ctw = jnp.dot(ct_ref[...], w1b_ref[...],
                  preferred_element_type=jnp.float32)
    onehot = (ids == lax.broadcasted_iota(jnp.int32, (RBP, NUM_CONFS), 1)
              ).astype(jnp.float32)
    hw = (jnp.dot(xb, w1a_ref[...], preferred_element_type=jnp.float32)
          + jnp.dot(onehot, ctw, preferred_element_type=jnp.float32))
    deg = degp_ref[0] + degp_ref[1] - 1.0
    dinv = lax.rsqrt(deg)
    g1_ref[...] = hw * dinv
    dinv_ref[...] = dinv


def _tc1(x, cid2, ct, w1a, w1b, degp):
    return pl.pallas_call(
        _tc1_body,
        grid=(GRIDP,),
        in_specs=[
            pl.BlockSpec((RBP, IN_DIM), lambda i: (i, 0)),
            pl.BlockSpec((RBP, 1), lambda i: (i, 0)),
            pl.BlockSpec((NUM_CONFS, CONF_EMB), lambda i: (0, 0)),
            pl.BlockSpec((IN_DIM, HIDDEN), lambda i: (0, 0)),
            pl.BlockSpec((CONF_EMB, HIDDEN), lambda i: (0, 0)),
            pl.BlockSpec((NCORES, RBP, 1), lambda i: (0, i, 0)),
        ],
        out_specs=[
            pl.BlockSpec((RBP, HIDDEN), lambda i: (i, 0)),
            pl.BlockSpec((RBP, 1), lambda i: (i, 0)),
        ],
        out_shape=[
            jax.ShapeDtypeStruct((NPAD, HIDDEN), jnp.float32),
            jax.ShapeDtypeStruct((NPAD, 1), jnp.float32),
        ],
    )(x, cid2, ct, w1a, w1b, degp)


def _tc2_body(p_ref, g1_ref, dinv_ref, b1_ref, w2_ref, g2_ref):
    agg = p_ref[0] + p_ref[1] - g1_ref[...]
    dinv = dinv_ref[...]
    h2 = jnp.maximum(agg * dinv + b1_ref[...], 0.0)
    hw2 = jnp.dot(h2, w2_ref[...], preferred_element_type=jnp.float32)
    g2_ref[...] = hw2 * dinv


def _tc2(p1, g1, dinv, b1r, w2):
    return pl.pallas_call(
        _tc2_body,
        grid=(GRIDP,),
        in_specs=[
            pl.BlockSpec((NCORES, RBP, HIDDEN), lambda i: (0, i, 0)),
            pl.BlockSpec((RBP, HIDDEN), lambda i: (i, 0)),
            pl.BlockSpec((RBP, 1), lambda i: (i, 0)),
            pl.BlockSpec((1, HIDDEN), lambda i: (0, 0)),
            pl.BlockSpec((HIDDEN, OUT), lambda i: (0, 0)),
        ],
        out_specs=pl.BlockSpec((RBP, OUT), lambda i: (i, 0)),
        out_shape=jax.ShapeDtypeStruct((NPAD, OUT), jnp.float32),
    )(p1, g1, dinv, b1r, w2)


RB3 = N
GRID3 = 1


def _tc3_body(q_ref, g2_ref, dinv_ref, b2_ref, out_ref):
    agg = q_ref[0] + q_ref[1] - g2_ref[...]
    out_ref[...] = agg * dinv_ref[...] + b2_ref[...]


def _tc3(p2, g2, dinv, b2r):
    return pl.pallas_call(
        _tc3_body,
        grid=(GRID3,),
        in_specs=[
            pl.BlockSpec((NCORES, RB3, OUT), lambda i: (0, i, 0)),
            pl.BlockSpec((RB3, OUT), lambda i: (i, 0)),
            pl.BlockSpec((RB3, 1), lambda i: (i, 0)),
            pl.BlockSpec((1, OUT), lambda i: (0, 0)),
        ],
        out_specs=pl.BlockSpec((RB3, OUT), lambda i: (i, 0)),
        out_shape=jax.ShapeDtypeStruct((N, OUT), jnp.float32),
    )(p2, g2, dinv, b2r)


@jax.jit
def _impl(x, conf_ids, edge_index, conf_table, W1, b1, W2, b2):
    ei = edge_index.astype(jnp.int32)
    cid2 = conf_ids.reshape(N, 1).astype(jnp.int32)
    ones128 = jnp.ones((CHUNK, 1), jnp.float32)

    degp = _sc_deg(ei, ones128)
    g1, dinv = _tc1(x, cid2, conf_table, W1[:IN_DIM], W1[IN_DIM:], degp)
    p1 = _sc_agg64(g1, ei)
    g2 = _tc2(p1, g1, dinv, b1.reshape(1, HIDDEN), W2)
    p2 = _sc_agg32(g2, ei)
    return _tc3(p2, g2, dinv, b2.reshape(1, OUT))


def kernel(x, conf_ids, edge_index, conf_table, W1, b1, W2, b2):
    return _impl(x, conf_ids, edge_index, conf_table, W1, b1, W2, b2)


# speedup vs baseline: 1.0649x; 1.0649x over previous
import functools

import jax
import jax.numpy as jnp
from jax import lax
from jax.experimental import pallas as pl
from jax.experimental.pallas import tpu as pltpu
from jax.experimental.pallas import tpu_sc as plsc

N = 10000
E = 320000
IN_DIM = 128
NUM_CONFS = 16
CONF_EMB = 4
HIDDEN = 64
OUT = 32

NPAD = 10240
NCORES = 2
NSUB = 16
NW = NCORES * NSUB
CHUNK = 128
EPW = E // NW
FULL = EPW // CHUNK
TAIL = EPW - FULL * CHUNK
SIDX = 10016
ROWS_PER_TILE = NPAD // NSUB

_mesh = lambda: plsc.VectorSubcoreMesh(core_axis_name="c", subcore_axis_name="s")
_SC_PARAMS = pltpu.CompilerParams(use_tc_tiling_on_sc=False)


@functools.partial(
    pl.kernel,
    out_type=jax.ShapeDtypeStruct((NCORES, NPAD, 1), jnp.float32),
    mesh=_mesh(),
    compiler_params=_SC_PARAMS,
    scratch_types=(
        [pltpu.VMEM_SHARED((NPAD, 1), jnp.float32),
         pltpu.VMEM((TAIL,), jnp.int32),
         pltpu.VMEM((CHUNK, 1), jnp.float32),
         pltpu.VMEM((TAIL, 1), jnp.float32)]
        + [pltpu.VMEM((CHUNK,), jnp.int32)] * 4
        + [pltpu.SemaphoreType.DMA] * 8
    ),
)
def _sc_deg(ei_hbm, ones_hbm, out_hbm, deg_s, dT, ones_v, ones_t, *bufs):
    dbufs = bufs[0:4]
    semd = bufs[4:8]
    sems = bufs[8:12]
    c = lax.axis_index("c")
    s = lax.axis_index("s")
    w = c * NSUB + s
    r0 = s * ROWS_PER_TILE
    base = w * EPW

    pltpu.sync_copy(ones_hbm, ones_v)
    pltpu.sync_copy(ones_hbm.at[pl.ds(0, TAIL)], ones_t)
    for i in range(ROWS_PER_TILE // CHUNK):
        pltpu.sync_copy(ones_v, deg_s.at[pl.ds(r0 + i * CHUNK, CHUNK)])

    plsc.subcore_barrier()

    def deg_group(j, nbuf):
        dds = [pltpu.async_copy(
                   ei_hbm.at[pl.ds(E + base + (j * 4 + k) * CHUNK, CHUNK)],
                   dbufs[k], semd[k])
               for k in range(nbuf)]
        sds = []
        for k in range(nbuf):
            dds[k].wait()
            sds.append(pltpu.async_copy(ones_v, deg_s.at[dbufs[k]], sems[k],
                                        add=True))
        for d in sds:
            d.wait()

    @pl.loop(0, FULL // 4)
    def _(j):
        deg_group(j, 4)

    deg_group(FULL // 4, FULL - (FULL // 4) * 4)

    pltpu.sync_copy(ei_hbm.at[pl.ds(E + base + FULL * CHUNK, TAIL)], dT)
    pltpu.sync_copy(ones_t, deg_s.at[dT], add=True)

    plsc.subcore_barrier()
    pltpu.sync_copy(deg_s.at[pl.ds(r0, ROWS_PER_TILE)],
                    out_hbm.at[c, pl.ds(r0, ROWS_PER_TILE)])


NBUF = 4
NGROUPS = FULL // NBUF
REM = FULL - NGROUPS * NBUF


def _make_sc_agg(D):
    @functools.partial(
        pl.kernel,
        out_type=jax.ShapeDtypeStruct((NCORES, NPAD, D), jnp.float32),
        mesh=_mesh(),
        compiler_params=_SC_PARAMS,
        scratch_types=(
            [pltpu.VMEM_SHARED((NPAD, D), jnp.float32),
             pltpu.VMEM_SHARED((NPAD, D), jnp.float32),
             pltpu.VMEM((SIDX,), jnp.int32),
             pltpu.VMEM((TAIL,), jnp.int32),
             pltpu.VMEM((TAIL, D), jnp.float32)]
            + [pltpu.VMEM((CHUNK,), jnp.int32)] * NBUF
            + [pltpu.VMEM((CHUNK, D), jnp.float32)] * NBUF
            + [pltpu.SemaphoreType.DMA] * (3 * NBUF)
        ),
    )
    def sc_agg(g_hbm, ei_hbm, out_hbm, agg_s, gtab_s, sidx, dtail, rtail,
               *bufs):
        dbufs = bufs[0:NBUF]
        rows = bufs[NBUF:2 * NBUF]
        semd = bufs[2 * NBUF:3 * NBUF]
        semg = bufs[3 * NBUF:4 * NBUF]
        sems = bufs[4 * NBUF:5 * NBUF]
        c = lax.axis_index("c")
        s = lax.axis_index("s")
        w = c * NSUB + s
        r0 = s * ROWS_PER_TILE
        base = w * EPW

        pltpu.sync_copy(ei_hbm.at[pl.ds(base, EPW)],
                        sidx.at[pl.ds(0, EPW)])
        pltpu.sync_copy(g_hbm.at[pl.ds(r0, ROWS_PER_TILE)],
                        gtab_s.at[pl.ds(r0, ROWS_PER_TILE)])
        pltpu.sync_copy(g_hbm.at[pl.ds(r0, ROWS_PER_TILE)],
                        agg_s.at[pl.ds(r0, ROWS_PER_TILE)])

        plsc.subcore_barrier()

        def do_group(j, nbuf):
            gds = []
            dds = []
            for k in range(nbuf):
                o = base + (j * NBUF + k) * CHUNK
                dds.append(pltpu.async_copy(ei_hbm.at[pl.ds(E + o, CHUNK)],
                                            dbufs[k], semd[k]))
                gds.append(pltpu.async_copy(
                    gtab_s.at[sidx.at[pl.ds((j * NBUF + k) * CHUNK, CHUNK)]],
                    rows[k], semg[k]))
            sds = []
            for k in range(nbuf):
                dds[k].wait()
                gds[k].wait()
                sds.append(pltpu.async_copy(rows[k], agg_s.at[dbufs[k]],
                                            sems[k], add=True))
            for d in sds:
                d.wait()

        @pl.loop(0, NGROUPS)
        def _(j):
            do_group(j, NBUF)

        do_group(NGROUPS, REM)

        pltpu.sync_copy(ei_hbm.at[pl.ds(E + base + FULL * CHUNK, TAIL)], dtail)
        pltpu.sync_copy(gtab_s.at[sidx.at[pl.ds(FULL * CHUNK, TAIL)]], rtail)
        pltpu.sync_copy(rtail, agg_s.at[dtail], add=True)

        plsc.subcore_barrier()
        pltpu.sync_copy(agg_s.at[pl.ds(r0, ROWS_PER_TILE)],
                        out_hbm.at[c, pl.ds(r0, ROWS_PER_TILE)])

    return sc_agg


_sc_agg64 = _make_sc_agg(HIDDEN)
_sc_agg32 = _make_sc_agg(OUT)


RBP = 2048
GRIDP = NPAD // RBP


def _tc1_body(x_ref, cid_ref, ct_ref, w1a_ref, w1b_ref, degp_ref,
              g1_ref, dinv_ref):
    xb = x_ref[...]
    ids = cid_ref[...]
    ctw = jnp.dot(ct_ref[...], w1b_ref[...],
                  preferred_element_type=jnp.float32)
    onehot = (ids == lax.broadcasted_iota(jnp.int32, (RBP, NUM_CONFS), 1)
              ).astype(jnp.float32)
    hw = (jnp.dot(xb, w1a_ref[...], preferred_element_type=jnp.float32)
          + jnp.dot(onehot, ctw, preferred_element_type=jnp.float32))
    deg = degp_ref[0] + degp_ref[1] - 1.0
    dinv = lax.rsqrt(deg)
    g1_ref[...] = hw * dinv
    dinv_ref[...] = dinv


def _tc1(x, cid2, ct, w1a, w1b, degp):
    return pl.pallas_call(
        _tc1_body,
        grid=(GRIDP,),
        in_specs=[
            pl.BlockSpec((RBP, IN_DIM), lambda i: (i, 0)),
            pl.BlockSpec((RBP, 1), lambda i: (i, 0)),
            pl.BlockSpec((NUM_CONFS, CONF_EMB), lambda i: (0, 0)),
            pl.BlockSpec((IN_DIM, HIDDEN), lambda i: (0, 0)),
            pl.BlockSpec((CONF_EMB, HIDDEN), lambda i: (0, 0)),
            pl.BlockSpec((NCORES, RBP, 1), lambda i: (0, i, 0)),
        ],
        out_specs=[
            pl.BlockSpec((RBP, HIDDEN), lambda i: (i, 0)),
            pl.BlockSpec((RBP, 1), lambda i: (i, 0)),
        ],
        out_shape=[
            jax.ShapeDtypeStruct((NPAD, HIDDEN), jnp.float32),
            jax.ShapeDtypeStruct((NPAD, 1), jnp.float32),
        ],
    )(x, cid2, ct, w1a, w1b, degp)


def _tc2_body(p_ref, g1_ref, dinv_ref, b1_ref, w2_ref, g2_ref):
    agg = p_ref[0] + p_ref[1] - g1_ref[...]
    dinv = dinv_ref[...]
    h2 = jnp.maximum(agg * dinv + b1_ref[...], 0.0)
    hw2 = jnp.dot(h2, w2_ref[...], preferred_element_type=jnp.float32)
    g2_ref[...] = hw2 * dinv


def _tc2(p1, g1, dinv, b1r, w2):
    return pl.pallas_call(
        _tc2_body,
        grid=(GRIDP,),
        in_specs=[
            pl.BlockSpec((NCORES, RBP, HIDDEN), lambda i: (0, i, 0)),
            pl.BlockSpec((RBP, HIDDEN), lambda i: (i, 0)),
            pl.BlockSpec((RBP, 1), lambda i: (i, 0)),
            pl.BlockSpec((1, HIDDEN), lambda i: (0, 0)),
            pl.BlockSpec((HIDDEN, OUT), lambda i: (0, 0)),
        ],
        out_specs=pl.BlockSpec((RBP, OUT), lambda i: (i, 0)),
        out_shape=jax.ShapeDtypeStruct((NPAD, OUT), jnp.float32),
    )(p1, g1, dinv, b1r, w2)


RB3 = 2000
GRID3 = N // RB3


def _tc3_body(q_ref, g2_ref, dinv_ref, b2_ref, out_ref):
    agg = q_ref[0] + q_ref[1] - g2_ref[...]
    out_ref[...] = agg * dinv_ref[...] + b2_ref[...]


def _tc3(p2, g2, dinv, b2r):
    return pl.pallas_call(
        _tc3_body,
        grid=(GRID3,),
        in_specs=[
            pl.BlockSpec((NCORES, RB3, OUT), lambda i: (0, i, 0)),
            pl.BlockSpec((RB3, OUT), lambda i: (i, 0)),
            pl.BlockSpec((RB3, 1), lambda i: (i, 0)),
            pl.BlockSpec((1, OUT), lambda i: (0, 0)),
        ],
        out_specs=pl.BlockSpec((RB3, OUT), lambda i: (i, 0)),
        out_shape=jax.ShapeDtypeStruct((N, OUT), jnp.float32),
    )(p2, g2, dinv, b2r)


@jax.jit
def _impl(x, conf_ids, edge_index, conf_table, W1, b1, W2, b2):
    ei = edge_index.astype(jnp.int32).reshape(2 * E)
    cid2 = conf_ids.reshape(N, 1).astype(jnp.int32)
    ones128 = jnp.ones((CHUNK, 1), jnp.float32)

    degp = _sc_deg(ei, ones128)
    g1, dinv = _tc1(x, cid2, conf_table, W1[:IN_DIM], W1[IN_DIM:], degp)
    p1 = _sc_agg64(g1, ei)
    g2 = _tc2(p1, g1, dinv, b1.reshape(1, HIDDEN), W2)
    p2 = _sc_agg32(g2, ei)
    return _tc3(p2, g2, dinv, b2.reshape(1, OUT))


def kernel(x, conf_ids, edge_index, conf_table, W1, b1, W2, b2):
    return _impl(x, conf_ids, edge_index, conf_table, W1, b1, W2, b2)
